# baseline (speedup 1.0000x reference)
import functools

import jax
import jax.numpy as jnp
from jax import lax
from jax.experimental import pallas as pl
from jax.experimental.pallas import tpu as pltpu
from jax.experimental.pallas import tpu_sc as plsc

N = 10000
E = 320000
D = 128
NG = 64
NC = 2
NS = 16
NW = NC * NS
CHUNK = 64
KBUF = 4
CB = 20
GPB = CB // KBUF
NSB = 8
NCHUNK = NSB * CB
EPW = NCHUNK * CHUNK
E_PAD = NW * EPW
DUMMY = 112
ROWS_PAD = N + DUMMY
RPS = ROWS_PAD // NS


def _sc_segment_sum(h, src_r, dst_r, zeros_hbm):
    mesh = plsc.VectorSubcoreMesh(core_axis_name="c", subcore_axis_name="s")

    row_types = [pltpu.VMEM((CHUNK, D), jnp.float32) for _ in range(KBUF)]
    idx_types = [pltpu.VMEM((CB, CHUNK), jnp.int32) for _ in range(4)]

    @functools.partial(
        pl.kernel,
        mesh=mesh,
        out_type=jax.ShapeDtypeStruct((NC, ROWS_PAD, D), jnp.float32),
        scratch_types=idx_types + row_types
        + [pltpu.VMEM_SHARED((ROWS_PAD, D), jnp.float32)]
        + [pltpu.SemaphoreType.DMA] * (2 * KBUF + 4),
    )
    def seg_sum(h_hbm, src_hbm, dst_hbm, z_hbm, out_hbm,
                src_a, dst_a, src_b, dst_b, *rest):
        rows = rest[:KBUF]
        acc = rest[KBUF]
        sems = rest[KBUF + 1:]
        gsems, ssems, tsems = sems[:KBUF], sems[KBUF:2 * KBUF], sems[2 * KBUF:]
        cid = lax.axis_index("c")
        sid = lax.axis_index("s")
        wid = cid * NS + sid

        def stage(s, src_v, dst_v, ts, td):
            pltpu.async_copy(src_hbm.at[wid, s], src_v, ts)
            pltpu.async_copy(dst_hbm.at[wid, s], dst_v, td)

        def stage_wait(src_v, dst_v, ts, td):
            pltpu.make_async_copy(src_hbm.at[wid, 0], src_v, ts).wait()
            pltpu.make_async_copy(dst_hbm.at[wid, 0], dst_v, td).wait()

        def drain(sem, b):
            pltpu.make_async_copy(h_hbm.at[pl.ds(0, CHUNK)], rows[b],
                                  sem).wait()

        def scat(dst_v, c, b):
            pltpu.async_copy(rows[b], acc.at[dst_v.at[c]], ssems[b], add=True)

        def gath(src_v, c, b):
            pltpu.async_copy(h_hbm.at[src_v.at[c]], rows[b], gsems[b])

        def process(src_x, dst_x, src_y, tail_wait):
            @pl.loop(0, GPB - 1)
            def _(g):
                base = g * KBUF
                for b in range(KBUF):
                    drain(gsems[b], b)
                    scat(dst_x, base + b, b)
                for b in range(KBUF):
                    drain(ssems[b], b)
                    gath(src_x, base + KBUF + b, b)
            for b in range(KBUF):
                drain(gsems[b], b)
                scat(dst_x, (GPB - 1) * KBUF + b, b)
            tail_wait()
            for b in range(KBUF):
                drain(ssems[b], b)
                gath(src_y, b, b)

        pltpu.sync_copy(z_hbm, acc.at[pl.ds(sid * RPS, RPS)])
        stage(0, src_a, dst_a, tsems[0], tsems[1])
        stage(1, src_b, dst_b, tsems[2], tsems[3])
        plsc.subcore_barrier()
        stage_wait(src_a, dst_a, tsems[0], tsems[1])
        for b in range(KBUF):
            gath(src_a, b, b)

        @pl.loop(0, NSB // 2)
        def _(t):
            process(src_a, dst_a, src_b,
                    lambda: stage_wait(src_b, dst_b, tsems[2], tsems[3]))

            @pl.when(2 * t + 2 < NSB)
            def _():
                stage(2 * t + 2, src_a, dst_a, tsems[0], tsems[1])

            def tail_b():
                @pl.when(2 * t + 2 < NSB)
                def _():
                    stage_wait(src_a, dst_a, tsems[0], tsems[1])

            process(src_b, dst_b, src_a, tail_b)

            @pl.when(2 * t + 3 < NSB)
            def _():
                stage(2 * t + 3, src_b, dst_b, tsems[2], tsems[3])

        for b in range(KBUF):
            drain(gsems[b], b)
        plsc.subcore_barrier()
        pltpu.sync_copy(acc.at[pl.ds(sid * RPS, RPS)],
                        out_hbm.at[cid, pl.ds(sid * RPS, RPS)])

    return seg_sum(h, src_r, dst_r, zeros_hbm)


_BR = 2000


def _layer_body(h_ref, p_ref, w_ref, b_ref, o_ref):
    z = h_ref[...] + p_ref[0] + p_ref[1]
    y = lax.dot_general(z, w_ref[...], (((1,), (1,)), ((), ())),
                        preferred_element_type=jnp.float32)
    o_ref[...] = jnp.maximum(y + b_ref[...], 0.0)


def _tc_layer(h, parts, W, b2):
    return pl.pallas_call(
        _layer_body,
        grid=(N // _BR,),
        in_specs=[
            pl.BlockSpec((_BR, D), lambda i: (i, 0)),
            pl.BlockSpec((NC, _BR, D), lambda i: (0, i, 0)),
            pl.BlockSpec((D, D), lambda i: (0, 0)),
            pl.BlockSpec((1, D), lambda i: (0, 0)),
        ],
        out_specs=pl.BlockSpec((_BR, D), lambda i: (i, 0)),
        out_shape=jax.ShapeDtypeStruct((N, D), jnp.float32),
    )(h, parts, W, b2)


def _pool_body(h_ref, batch_ref, w_ref, b_ref, o_ref):
    ids = lax.broadcasted_iota(jnp.int32, (NG, N), 0)
    mask = (ids == batch_ref[...]).astype(jnp.float32)
    sums = lax.dot_general(mask, h_ref[...], (((1,), (0,)), ((), ())),
                           preferred_element_type=jnp.float32)
    counts = jnp.sum(mask, axis=1, keepdims=True)
    pooled = sums / jnp.maximum(counts, 1.0)
    y = lax.dot_general(pooled, w_ref[...], (((1,), (1,)), ((), ())),
                        preferred_element_type=jnp.float32)
    o_ref[...] = y + b_ref[...]


def _tc_pool_fc(h, batch2, Wfc, bfc2):
    return pl.pallas_call(
        _pool_body,
        out_shape=jax.ShapeDtypeStruct((NG, D), jnp.float32),
    )(h, batch2, Wfc, bfc2)


def kernel(x, edge_index, batch, W1, b1, W2, b2, W3, b3, Wfc, bfc):
    src = edge_index[0].astype(jnp.int32)
    dst = edge_index[1].astype(jnp.int32)
    n_pad = E_PAD - E
    pad_src = jnp.zeros((n_pad,), jnp.int32)
    pad_dst = N + (jnp.arange(n_pad, dtype=jnp.int32) % DUMMY)
    src_r = jnp.concatenate([src, pad_src]).reshape(NW, NSB, CB, CHUNK)
    dst_r = jnp.concatenate([dst, pad_dst]).reshape(NW, NSB, CB, CHUNK)
    zeros_hbm = jnp.zeros((RPS, D), jnp.float32)
    batch2 = batch.astype(jnp.int32).reshape(1, N)

    h = x
    for W, b in ((W1, b1), (W2, b2), (W3, b3)):
        parts = _sc_segment_sum(h, src_r, dst_r, zeros_hbm)
        h = _tc_layer(h, parts, W, b.reshape(1, D))
    return _tc_pool_fc(h, batch2, Wfc, bfc.reshape(1, D))


# speedup vs baseline: 1.1434x; 1.1434x over previous
import functools

import jax
import jax.numpy as jnp
from jax import lax
from jax.experimental import pallas as pl
from jax.experimental.pallas import tpu as pltpu
from jax.experimental.pallas import tpu_sc as plsc

N = 10000
E = 320000
D = 128
NG = 64
NC = 2
NS = 16
NW = NC * NS
CHUNK = 64
EPW = 10240
EPW_REAL = E // NW
CB = 16
NSB = 10
E_PAD = NW * EPW
KBUF = 4
DUMMY = 240
ROWS_PAD = N + DUMMY
RPS = ROWS_PAD // NS


def _sc_segment_sum(h, src_r, dst_r, zeros_hbm):
    mesh = plsc.VectorSubcoreMesh(core_axis_name="c", subcore_axis_name="s")

    @functools.partial(
        pl.kernel,
        mesh=mesh,
        out_type=jax.ShapeDtypeStruct((NC, ROWS_PAD, D), jnp.float32),
        scratch_types=[
            pltpu.VMEM((CB, CHUNK), jnp.int32),
            pltpu.VMEM((CB, CHUNK), jnp.int32),
            pltpu.VMEM((KBUF, CHUNK, D), jnp.float32),
            pltpu.VMEM_SHARED((ROWS_PAD, D), jnp.float32),
        ] + [pltpu.SemaphoreType.DMA] * (2 * KBUF),
    )
    def seg_sum(h_hbm, src_hbm, dst_hbm, z_hbm, out_hbm, src_v, dst_v, rows_v,
                acc, *sems):
        gsems, ssems = sems[:KBUF], sems[KBUF:]
        cid = lax.axis_index("c")
        sid = lax.axis_index("s")
        wid = cid * NS + sid
        pltpu.sync_copy(z_hbm, acc.at[pl.ds(sid * RPS, RPS)])
        plsc.subcore_barrier()

        @pl.loop(0, NSB)
        def _(s):
            pltpu.sync_copy(src_hbm.at[wid, s], src_v)
            pltpu.sync_copy(dst_hbm.at[wid, s], dst_v)

            @pl.loop(0, CB, step=KBUF)
            def _(j):
                gets = [pltpu.async_copy(h_hbm.at[src_v.at[j + b]],
                                         rows_v.at[b], gsems[b])
                        for b in range(KBUF)]
                puts = []
                for b in range(KBUF):
                    gets[b].wait()
                    puts.append(pltpu.async_copy(rows_v.at[b],
                                                 acc.at[dst_v.at[j + b]],
                                                 ssems[b], add=True))
                for p in puts:
                    p.wait()

        plsc.subcore_barrier()
        pltpu.sync_copy(acc.at[pl.ds(sid * RPS, RPS)],
                        out_hbm.at[cid, pl.ds(sid * RPS, RPS)])

    return seg_sum(h, src_r, dst_r, zeros_hbm)


_BR = 2000


def _layer_body(h_ref, p_ref, w_ref, b_ref, o_ref):
    z = h_ref[...] + p_ref[0] + p_ref[1]
    y = lax.dot_general(z, w_ref[...], (((1,), (1,)), ((), ())),
                        preferred_element_type=jnp.float32)
    o_ref[...] = jnp.maximum(y + b_ref[...], 0.0)


def _tc_layer(h, parts, W, b2):
    return pl.pallas_call(
        _layer_body,
        grid=(N // _BR,),
        in_specs=[
            pl.BlockSpec((_BR, D), lambda i: (i, 0)),
            pl.BlockSpec((NC, _BR, D), lambda i: (0, i, 0)),
            pl.BlockSpec((D, D), lambda i: (0, 0)),
            pl.BlockSpec((1, D), lambda i: (0, 0)),
        ],
        out_specs=pl.BlockSpec((_BR, D), lambda i: (i, 0)),
        out_shape=jax.ShapeDtypeStruct((N, D), jnp.float32),
    )(h, parts, W, b2)


def _pool_body(h_ref, batch_ref, w_ref, b_ref, o_ref):
    ids = lax.broadcasted_iota(jnp.int32, (NG, N), 0)
    mask = (ids == batch_ref[...]).astype(jnp.float32)
    sums = lax.dot_general(mask, h_ref[...], (((1,), (0,)), ((), ())),
                           preferred_element_type=jnp.float32)
    counts = jnp.sum(mask, axis=1, keepdims=True)
    pooled = sums / jnp.maximum(counts, 1.0)
    y = lax.dot_general(pooled, w_ref[...], (((1,), (1,)), ((), ())),
                        preferred_element_type=jnp.float32)
    o_ref[...] = y + b_ref[...]


def _tc_pool_fc(h, batch2, Wfc, bfc2):
    return pl.pallas_call(
        _pool_body,
        out_shape=jax.ShapeDtypeStruct((NG, D), jnp.float32),
    )(h, batch2, Wfc, bfc2)


def kernel(x, edge_index, batch, W1, b1, W2, b2, W3, b3, Wfc, bfc):
    src = edge_index[0].astype(jnp.int32).reshape(NW, EPW_REAL)
    dst = edge_index[1].astype(jnp.int32).reshape(NW, EPW_REAL)
    n_pad = EPW - EPW_REAL
    pad_src = jnp.zeros((NW, n_pad), jnp.int32)
    pad_dst = jnp.broadcast_to(
        N + jnp.arange(n_pad, dtype=jnp.int32) % DUMMY, (NW, n_pad))
    src_r = jnp.concatenate([src, pad_src], axis=1).reshape(NW, NSB, CB, CHUNK)
    dst_r = jnp.concatenate([dst, pad_dst], axis=1).reshape(NW, NSB, CB, CHUNK)
    zeros_hbm = jnp.zeros((RPS, D), jnp.float32)
    batch2 = batch.astype(jnp.int32).reshape(1, N)

    h = x
    for W, b in ((W1, b1), (W2, b2), (W3, b3)):
        parts = _sc_segment_sum(h, src_r, dst_r, zeros_hbm)
        h = _tc_layer(h, parts, W, b.reshape(1, D))
    return _tc_pool_fc(h, batch2, Wfc, bfc.reshape(1, D))


# speedup vs baseline: 1.3114x; 1.1469x over previous
import functools

import jax
import jax.numpy as jnp
from jax import lax
from jax.experimental import pallas as pl
from jax.experimental.pallas import tpu as pltpu
from jax.experimental.pallas import tpu_sc as plsc

N = 10000
E = 320000
D = 128
NG = 64
NC = 2
NS = 16
NW = NC * NS
CHUNK = 64
EPW = 10240
EPW_REAL = E // NW
CB = 16
NSB = 10
E_PAD = NW * EPW
KBUF = 4
DUMMY = 240
ROWS_PAD = N + DUMMY
RPS = ROWS_PAD // NS


def _sc_segment_sum(h, src_r, dst_r, zeros_hbm):
    mesh = plsc.VectorSubcoreMesh(core_axis_name="c", subcore_axis_name="s")

    @functools.partial(
        pl.kernel,
        mesh=mesh,
        out_type=jax.ShapeDtypeStruct((NC, ROWS_PAD, D), jnp.float32),
        scratch_types=[
            pltpu.VMEM((CB, CHUNK), jnp.int32),
            pltpu.VMEM((CB, CHUNK), jnp.int32),
            pltpu.VMEM((KBUF, CHUNK, D), jnp.float32),
            pltpu.VMEM_SHARED((ROWS_PAD, D), jnp.float32),
        ] + [pltpu.SemaphoreType.DMA] * (2 * KBUF),
    )
    def seg_sum(h_hbm, src_hbm, dst_hbm, z_hbm, out_hbm, src_v, dst_v, rows_v,
                acc, *sems):
        gsems, ssems = sems[:KBUF], sems[KBUF:]
        cid = lax.axis_index("c")
        sid = lax.axis_index("s")
        wid = cid * NS + sid
        pltpu.sync_copy(z_hbm, acc.at[pl.ds(sid * RPS, RPS)])
        plsc.subcore_barrier()

        @pl.loop(0, NSB)
        def _(s):
            pltpu.sync_copy(src_hbm.at[wid, s], src_v)
            pltpu.sync_copy(dst_hbm.at[wid, s], dst_v)

            @pl.loop(0, CB, step=KBUF)
            def _(j):
                gets = [pltpu.async_copy(h_hbm.at[src_v.at[j + b]],
                                         rows_v.at[b], gsems[b])
                        for b in range(KBUF)]
                puts = []
                for b in range(KBUF):
                    gets[b].wait()
                    puts.append(pltpu.async_copy(rows_v.at[b],
                                                 acc.at[dst_v.at[j + b]],
                                                 ssems[b], add=True))
                for p in puts:
                    p.wait()

        plsc.subcore_barrier()
        pltpu.sync_copy(acc.at[pl.ds(sid * RPS, RPS)],
                        out_hbm.at[cid, pl.ds(sid * RPS, RPS)])

    return seg_sum(h, src_r, dst_r, zeros_hbm)


_BR = 2000


def _layer_body(h_ref, p_ref, w_ref, b_ref, o_ref):
    z = h_ref[...] + p_ref[0] + p_ref[1]
    y = lax.dot_general(z, w_ref[...], (((1,), (1,)), ((), ())),
                        preferred_element_type=jnp.float32)
    o_ref[...] = jnp.maximum(y + b_ref[...], 0.0)


def _tc_layer(h, parts, W, b2):
    return pl.pallas_call(
        _layer_body,
        grid=(N // _BR,),
        in_specs=[
            pl.BlockSpec((_BR, D), lambda i: (i, 0)),
            pl.BlockSpec((NC, _BR, D), lambda i: (0, i, 0)),
            pl.BlockSpec((D, D), lambda i: (0, 0)),
            pl.BlockSpec((1, D), lambda i: (0, 0)),
        ],
        out_specs=pl.BlockSpec((_BR, D), lambda i: (i, 0)),
        out_shape=jax.ShapeDtypeStruct((N, D), jnp.float32),
    )(h, parts, W, b2)


def _pool_body(h_ref, batch_ref, w_ref, b_ref, o_ref):
    ids = lax.broadcasted_iota(jnp.int32, (NG, N), 0)
    mask = (ids == batch_ref[...]).astype(jnp.float32)
    sums = lax.dot_general(mask, h_ref[...], (((1,), (0,)), ((), ())),
                           preferred_element_type=jnp.float32)
    counts = jnp.sum(mask, axis=1, keepdims=True)
    pooled = sums / jnp.maximum(counts, 1.0)
    y = lax.dot_general(pooled, w_ref[...], (((1,), (1,)), ((), ())),
                        preferred_element_type=jnp.float32)
    o_ref[...] = y + b_ref[...]


def _tc_pool_fc(h, batch2, Wfc, bfc2):
    return pl.pallas_call(
        _pool_body,
        out_shape=jax.ShapeDtypeStruct((NG, D), jnp.float32),
    )(h, batch2, Wfc, bfc2)


def kernel(x, edge_index, batch, W1, b1, W2, b2, W3, b3, Wfc, bfc):
    src = edge_index[0].astype(jnp.int32).reshape(NW, 40, 250)
    dst = edge_index[1].astype(jnp.int32).reshape(NW, 40, 250)
    pad_src = jnp.zeros((NW, 40, 6), jnp.int32)
    s_ids = (jnp.arange(NW, dtype=jnp.int32) % NS)[:, None]
    pad_rows = N + 15 * s_ids + (jnp.arange(240, dtype=jnp.int32) % 15)[None]
    pad_dst = jnp.broadcast_to(pad_rows.reshape(NW, 40, 6), (NW, 40, 6))
    src_r = jnp.concatenate([src, pad_src], axis=2).reshape(NW, NSB, CB, CHUNK)
    dst_r = jnp.concatenate([dst, pad_dst], axis=2).reshape(NW, NSB, CB, CHUNK)
    zeros_hbm = jnp.zeros((RPS, D), jnp.float32)
    batch2 = batch.astype(jnp.int32).reshape(1, N)

    h = x
    for W, b in ((W1, b1), (W2, b2), (W3, b3)):
        parts = _sc_segment_sum(h, src_r, dst_r, zeros_hbm)
        h = _tc_layer(h, parts, W, b.reshape(1, D))
    return _tc_pool_fc(h, batch2, Wfc, bfc.reshape(1, D))


# speedup vs baseline: 1.3255x; 1.0108x over previous
import functools

import jax
import jax.numpy as jnp
from jax import lax
from jax.experimental import pallas as pl
from jax.experimental.pallas import tpu as pltpu
from jax.experimental.pallas import tpu_sc as plsc

N = 10000
E = 320000
D = 128
NG = 64
NC = 2
NS = 16
NW = NC * NS
CHUNK = 64
EPW = 10240
EPW_REAL = E // NW
CB = 20
NSB = 8
E_PAD = NW * EPW
KBUF = 5
DUMMY = 240
ROWS_PAD = N + DUMMY
RPS = ROWS_PAD // NS


def _sc_segment_sum(h, src_r, dst_r, zeros_hbm):
    mesh = plsc.VectorSubcoreMesh(core_axis_name="c", subcore_axis_name="s")

    @functools.partial(
        pl.kernel,
        mesh=mesh,
        out_type=jax.ShapeDtypeStruct((NC, ROWS_PAD, D), jnp.float32),
        scratch_types=[
            pltpu.VMEM((CB, CHUNK), jnp.int32),
            pltpu.VMEM((CB, CHUNK), jnp.int32),
        ] + [pltpu.VMEM((CHUNK, D), jnp.float32) for _ in range(KBUF)]
        + [
            pltpu.VMEM_SHARED((ROWS_PAD, D), jnp.float32),
        ] + [pltpu.SemaphoreType.DMA] * (2 * KBUF),
    )
    def seg_sum(h_hbm, src_hbm, dst_hbm, z_hbm, out_hbm, src_v, dst_v, *rest):
        rows_v = rest[:KBUF]
        acc = rest[KBUF]
        sems = rest[KBUF + 1:]
        gsems, ssems = sems[:KBUF], sems[KBUF:]
        cid = lax.axis_index("c")
        sid = lax.axis_index("s")
        wid = cid * NS + sid
        pltpu.sync_copy(z_hbm, acc.at[pl.ds(sid * RPS, RPS)])
        plsc.subcore_barrier()

        @pl.loop(0, NSB)
        def _(s):
            pltpu.sync_copy(src_hbm.at[wid, s], src_v)
            pltpu.sync_copy(dst_hbm.at[wid, s], dst_v)

            @pl.loop(0, CB, step=KBUF)
            def _(j):
                gets = [pltpu.async_copy(h_hbm.at[src_v.at[j + b]],
                                         rows_v[b], gsems[b])
                        for b in range(KBUF)]
                puts = []
                for b in range(KBUF):
                    gets[b].wait()
                    puts.append(pltpu.async_copy(rows_v[b],
                                                 acc.at[dst_v.at[j + b]],
                                                 ssems[b], add=True))
                for p in puts:
                    p.wait()

        plsc.subcore_barrier()
        pltpu.sync_copy(acc.at[pl.ds(sid * RPS, RPS)],
                        out_hbm.at[cid, pl.ds(sid * RPS, RPS)])

    return seg_sum(h, src_r, dst_r, zeros_hbm)


_BR = 2000


def _layer_body(h_ref, p_ref, w_ref, b_ref, o_ref):
    z = h_ref[...] + p_ref[0] + p_ref[1]
    y = lax.dot_general(z, w_ref[...], (((1,), (1,)), ((), ())),
                        preferred_element_type=jnp.float32)
    o_ref[...] = jnp.maximum(y + b_ref[...], 0.0)


def _tc_layer(h, parts, W, b2):
    return pl.pallas_call(
        _layer_body,
        grid=(N // _BR,),
        in_specs=[
            pl.BlockSpec((_BR, D), lambda i: (i, 0)),
            pl.BlockSpec((NC, _BR, D), lambda i: (0, i, 0)),
            pl.BlockSpec((D, D), lambda i: (0, 0)),
            pl.BlockSpec((1, D), lambda i: (0, 0)),
        ],
        out_specs=pl.BlockSpec((_BR, D), lambda i: (i, 0)),
        out_shape=jax.ShapeDtypeStruct((N, D), jnp.float32),
    )(h, parts, W, b2)


def _pool_body(h_ref, batch_ref, w_ref, b_ref, o_ref):
    ids = lax.broadcasted_iota(jnp.int32, (NG, N), 0)
    mask = (ids == batch_ref[...]).astype(jnp.float32)
    sums = lax.dot_general(mask, h_ref[...], (((1,), (0,)), ((), ())),
                           preferred_element_type=jnp.float32)
    counts = jnp.sum(mask, axis=1, keepdims=True)
    pooled = sums / jnp.maximum(counts, 1.0)
    y = lax.dot_general(pooled, w_ref[...], (((1,), (1,)), ((), ())),
                        preferred_element_type=jnp.float32)
    o_ref[...] = y + b_ref[...]


def _tc_pool_fc(h, batch2, Wfc, bfc2):
    return pl.pallas_call(
        _pool_body,
        out_shape=jax.ShapeDtypeStruct((NG, D), jnp.float32),
    )(h, batch2, Wfc, bfc2)


def kernel(x, edge_index, batch, W1, b1, W2, b2, W3, b3, Wfc, bfc):
    src = edge_index[0].astype(jnp.int32).reshape(NW, 40, 250)
    dst = edge_index[1].astype(jnp.int32).reshape(NW, 40, 250)
    pad_src = jnp.zeros((NW, 40, 6), jnp.int32)
    s_ids = (jnp.arange(NW, dtype=jnp.int32) % NS)[:, None]
    pad_rows = N + 15 * s_ids + (jnp.arange(240, dtype=jnp.int32) % 15)[None]
    pad_dst = jnp.broadcast_to(pad_rows.reshape(NW, 40, 6), (NW, 40, 6))
    src_r = jnp.concatenate([src, pad_src], axis=2).reshape(NW, NSB, CB, CHUNK)
    dst_r = jnp.concatenate([dst, pad_dst], axis=2).reshape(NW, NSB, CB, CHUNK)
    zeros_hbm = jnp.zeros((RPS, D), jnp.float32)
    batch2 = batch.astype(jnp.int32).reshape(1, N)

    h = x
    for W, b in ((W1, b1), (W2, b2), (W3, b3)):
        parts = _sc_segment_sum(h, src_r, dst_r, zeros_hbm)
        h = _tc_layer(h, parts, W, b.reshape(1, D))
    return _tc_pool_fc(h, batch2, Wfc, bfc.reshape(1, D))
